# baseline (speedup 1.0000x reference)
import functools

import jax
import jax.numpy as jnp
from jax import lax
from jax.experimental import pallas as pl
from jax.experimental.pallas import tpu as pltpu
from jax.experimental.pallas import tpu_sc as plsc

BATCH = 4096
DIM = 64
NUM_TABLES = 6

_info = plsc.get_sparse_core_info()
_NC, _NS = _info.num_cores, _info.num_subcores
_NW = _NC * _NS
_BPW = BATCH // _NW


def _make_sc_lookup():
  mesh = plsc.VectorSubcoreMesh(core_axis_name="c", subcore_axis_name="s")

  @functools.partial(
      pl.kernel,
      mesh=mesh,
      out_type=jax.ShapeDtypeStruct((BATCH, NUM_TABLES, DIM), jnp.float32),
      compiler_params=pltpu.CompilerParams(use_tc_tiling_on_sc=False),
      scratch_types=[
          pltpu.VMEM((_BPW,), jnp.int32),
          pltpu.VMEM((_BPW,), jnp.int32),
          pltpu.VMEM((NUM_TABLES, _BPW, DIM), jnp.float32),
          pltpu.SemaphoreType.DMA,
          pltpu.SemaphoreType.DMA,
      ],
  )
  def lookup(u_hbm, i_hbm, t0, t1, t2, t3, t4, t5, out_hbm,
             uid_v, iid_v, rows_v, gsem, wsem):
    wid = lax.axis_index("s") * _NC + lax.axis_index("c")
    base = wid * _BPW
    pltpu.sync_copy(u_hbm.at[pl.ds(base, _BPW)], uid_v)
    pltpu.sync_copy(i_hbm.at[pl.ds(base, _BPW)], iid_v)
    tables = (t0, t1, t2, t3, t4, t5)
    for k in range(NUM_TABLES):
      idx = uid_v if k % 2 == 0 else iid_v
      pltpu.async_copy(tables[k].at[idx], rows_v.at[k], gsem)
    for k in range(NUM_TABLES):
      idx = uid_v if k % 2 == 0 else iid_v
      pltpu.make_async_copy(tables[k].at[idx], rows_v.at[k], gsem).wait()
      pltpu.async_copy(rows_v.at[k], out_hbm.at[pl.ds(base, _BPW), k, :],
                       wsem)
    for k in range(NUM_TABLES):
      pltpu.make_async_copy(rows_v.at[k],
                            out_hbm.at[pl.ds(base, _BPW), k, :], wsem).wait()

  return lookup


_sc_lookup = _make_sc_lookup()


def kernel(uid, iid, user_table, item_table, src_user_0, src_item_0,
           src_user_1, src_item_1):
  out3 = _sc_lookup(uid.astype(jnp.int32), iid.astype(jnp.int32),
                    user_table, item_table, src_user_0, src_item_0,
                    src_user_1, src_item_1)
  return out3.reshape(BATCH, NUM_TABLES * DIM)


# speedup vs baseline: 1.5255x; 1.5255x over previous
import functools

import jax
import jax.numpy as jnp
from jax import lax
from jax.experimental import pallas as pl
from jax.experimental.pallas import tpu as pltpu
from jax.experimental.pallas import tpu_sc as plsc

BATCH = 4096
DIM = 64
NUM_TABLES = 6

_info = plsc.get_sparse_core_info()
_NC, _NS = _info.num_cores, _info.num_subcores
_NW = _NC * _NS
_BPW = BATCH // _NW


def _make():
  mesh = plsc.VectorSubcoreMesh(core_axis_name="c", subcore_axis_name="s")

  @functools.partial(
      pl.kernel,
      mesh=mesh,
      out_type=jax.ShapeDtypeStruct((BATCH, NUM_TABLES * DIM), jnp.float32),
      compiler_params=pltpu.CompilerParams(use_tc_tiling_on_sc=True),
      scratch_types=[
          pltpu.VMEM((_BPW,), jnp.int32),
          pltpu.VMEM((_BPW,), jnp.int32),
          pltpu.VMEM((NUM_TABLES // 2, _BPW, 2 * DIM), jnp.float32),
          pltpu.SemaphoreType.DMA,
          pltpu.SemaphoreType.DMA,
      ],
  )
  def lookup(u_hbm, i_hbm, t0, t1, t2, t3, t4, t5, out_hbm,
             uid_v, iid_v, gbuf, gsem, wsem):
    wid = lax.axis_index("s") * _NC + lax.axis_index("c")
    base = wid * _BPW
    pltpu.sync_copy(u_hbm.at[pl.ds(base, _BPW)], uid_v)
    pltpu.sync_copy(i_hbm.at[pl.ds(base, _BPW)], iid_v)
    tables = (t0, t1, t2, t3, t4, t5)

    def body(c, carry):
      uv = uid_v[pl.ds(c * 16, 16)]
      iv = iid_v[pl.ds(c * 16, 16)]
      for j in range(16):
        r = c * 16 + j
        for k in range(NUM_TABLES):
          idx = uv[j] if k % 2 == 0 else iv[j]
          pltpu.async_copy(
              tables[k].at[idx],
              gbuf.at[k // 2, r, pl.ds((k % 2) * DIM, DIM)], gsem)
      for j in range(16):
        r = c * 16 + j
        for k in range(NUM_TABLES):
          idx = uv[j] if k % 2 == 0 else iv[j]
          pltpu.make_async_copy(
              tables[k].at[idx],
              gbuf.at[k // 2, r, pl.ds((k % 2) * DIM, DIM)], gsem).wait()
      return carry

    lax.fori_loop(0, _BPW // 16, body, 0)

    for p in range(NUM_TABLES // 2):
      pltpu.async_copy(
          gbuf.at[p], out_hbm.at[pl.ds(base, _BPW), pl.ds(p * 2 * DIM, 2 * DIM)],
          wsem)
    for p in range(NUM_TABLES // 2):
      pltpu.make_async_copy(
          gbuf.at[p], out_hbm.at[pl.ds(base, _BPW), pl.ds(p * 2 * DIM, 2 * DIM)],
          wsem).wait()

  return lookup


_lookup = _make()


def kernel(uid, iid, user_table, item_table, src_user_0, src_item_0,
           src_user_1, src_item_1):
  return _lookup(uid.astype(jnp.int32), iid.astype(jnp.int32),
                 user_table, item_table, src_user_0, src_item_0,
                 src_user_1, src_item_1)


# speedup vs baseline: 1.5526x; 1.0178x over previous
import functools

import jax
import jax.numpy as jnp
from jax import lax
from jax.experimental import pallas as pl
from jax.experimental.pallas import tpu as pltpu
from jax.experimental.pallas import tpu_sc as plsc

BATCH = 4096
DIM = 64
NUM_TABLES = 6

_info = plsc.get_sparse_core_info()
_NC, _NS = _info.num_cores, _info.num_subcores
_NW = _NC * _NS
_BPW = BATCH // _NW


def _make():
  mesh = plsc.VectorSubcoreMesh(core_axis_name="c", subcore_axis_name="s")

  @functools.partial(
      pl.kernel,
      mesh=mesh,
      out_type=jax.ShapeDtypeStruct((BATCH, NUM_TABLES * DIM), jnp.float32),
      compiler_params=pltpu.CompilerParams(use_tc_tiling_on_sc=True),
      scratch_types=[
          pltpu.VMEM((_BPW,), jnp.int32),
          pltpu.VMEM((_BPW,), jnp.int32),
          pltpu.VMEM((NUM_TABLES // 2, _BPW, 2 * DIM), jnp.float32),
          pltpu.SemaphoreType.DMA,
          pltpu.SemaphoreType.DMA,
      ],
  )
  def lookup(u_hbm, i_hbm, t0, t1, t2, t3, t4, t5, out_hbm,
             uid_v, iid_v, gbuf, gsem, wsem):
    wid = lax.axis_index("s") * _NC + lax.axis_index("c")
    base = wid * _BPW
    pltpu.sync_copy(u_hbm.at[pl.ds(base, _BPW)], uid_v)
    pltpu.sync_copy(i_hbm.at[pl.ds(base, _BPW)], iid_v)
    tables = (t0, t1, t2, t3, t4, t5)

    def body(c, carry):
      uv = uid_v[pl.ds(c * 16, 16)]
      iv = iid_v[pl.ds(c * 16, 16)]
      for j in range(16):
        r = c * 16 + j
        for k in range(NUM_TABLES):
          idx = uv[j] if k % 2 == 0 else iv[j]
          pltpu.async_copy(
              tables[k].at[idx],
              gbuf.at[k // 2, r, pl.ds((k % 2) * DIM, DIM)], gsem)
      return carry

    lax.fori_loop(0, _BPW // 16, body, 0)

    for p in range(NUM_TABLES // 2):
      pltpu.make_async_copy(
          out_hbm.at[pl.ds(base, _BPW), pl.ds(p * 2 * DIM, 2 * DIM)],
          gbuf.at[p], gsem).wait()

    for p in range(NUM_TABLES // 2):
      pltpu.async_copy(
          gbuf.at[p],
          out_hbm.at[pl.ds(base, _BPW), pl.ds(p * 2 * DIM, 2 * DIM)], wsem)
    for p in range(NUM_TABLES // 2):
      pltpu.make_async_copy(
          gbuf.at[p],
          out_hbm.at[pl.ds(base, _BPW), pl.ds(p * 2 * DIM, 2 * DIM)],
          wsem).wait()

  return lookup


_lookup = _make()


def kernel(uid, iid, user_table, item_table, src_user_0, src_item_0,
           src_user_1, src_item_1):
  return _lookup(uid.astype(jnp.int32), iid.astype(jnp.int32),
                 user_table, item_table, src_user_0, src_item_0,
                 src_user_1, src_item_1)


# speedup vs baseline: 1.5601x; 1.0048x over previous
import functools

import jax
import jax.numpy as jnp
from jax import lax
from jax.experimental import pallas as pl
from jax.experimental.pallas import tpu as pltpu
from jax.experimental.pallas import tpu_sc as plsc

BATCH = 4096
DIM = 64
NUM_TABLES = 6

_info = plsc.get_sparse_core_info()
_NC, _NS = _info.num_cores, _info.num_subcores
_NW = _NC * _NS
_BPW = BATCH // _NW


def _make():
  mesh = plsc.VectorSubcoreMesh(core_axis_name="c", subcore_axis_name="s")

  @functools.partial(
      pl.kernel,
      mesh=mesh,
      out_type=jax.ShapeDtypeStruct((BATCH, NUM_TABLES * DIM), jnp.float32),
      compiler_params=pltpu.CompilerParams(use_tc_tiling_on_sc=True),
      scratch_types=[
          pltpu.VMEM((_BPW,), jnp.int32),
          pltpu.VMEM((_BPW,), jnp.int32),
          pltpu.VMEM((NUM_TABLES // 2, _BPW, 2 * DIM), jnp.float32),
          pltpu.SemaphoreType.DMA,
          pltpu.SemaphoreType.DMA,
          pltpu.SemaphoreType.DMA,
          pltpu.SemaphoreType.DMA,
          pltpu.SemaphoreType.DMA,
          pltpu.SemaphoreType.DMA,
          pltpu.SemaphoreType.DMA,
      ],
  )
  def lookup(u_hbm, i_hbm, t0, t1, t2, t3, t4, t5, out_hbm,
             uid_v, iid_v, gbuf, g0, g1, g2, g3, g4, g5, wsem):
    gsems = (g0, g1, g2, g3, g4, g5)
    wid = lax.axis_index("s") * _NC + lax.axis_index("c")
    base = wid * _BPW
    pltpu.sync_copy(u_hbm.at[pl.ds(base, _BPW)], uid_v)
    pltpu.sync_copy(i_hbm.at[pl.ds(base, _BPW)], iid_v)
    tables = (t0, t1, t2, t3, t4, t5)

    def body(c, carry):
      uv = uid_v[pl.ds(c * 16, 16)]
      iv = iid_v[pl.ds(c * 16, 16)]
      for j in range(16):
        r = c * 16 + j
        for k in range(NUM_TABLES):
          idx = uv[j] if k % 2 == 0 else iv[j]
          pltpu.async_copy(
              tables[k].at[idx],
              gbuf.at[k // 2, r, pl.ds((k % 2) * DIM, DIM)], gsems[k])
      return carry

    lax.fori_loop(0, _BPW // 16, body, 0)

    for k in range(NUM_TABLES):
      pltpu.make_async_copy(
          out_hbm.at[pl.ds(base, _BPW // 2), pl.ds((k // 2) * 2 * DIM, 2 * DIM)],
          gbuf.at[k // 2, pl.ds(0, _BPW // 2)], gsems[k]).wait()

    for p in range(NUM_TABLES // 2):
      pltpu.async_copy(
          gbuf.at[p],
          out_hbm.at[pl.ds(base, _BPW), pl.ds(p * 2 * DIM, 2 * DIM)], wsem)
    for p in range(NUM_TABLES // 2):
      pltpu.make_async_copy(
          gbuf.at[p],
          out_hbm.at[pl.ds(base, _BPW), pl.ds(p * 2 * DIM, 2 * DIM)],
          wsem).wait()

  return lookup


_lookup = _make()


def kernel(uid, iid, user_table, item_table, src_user_0, src_item_0,
           src_user_1, src_item_1):
  return _lookup(uid.astype(jnp.int32), iid.astype(jnp.int32),
                 user_table, item_table, src_user_0, src_item_0,
                 src_user_1, src_item_1)
